# baseline (speedup 1.0000x reference)
import functools

import jax
import jax.numpy as jnp
from jax import lax
from jax.experimental import pallas as pl
from jax.experimental.pallas import tpu as pltpu
from jax.experimental.pallas import tpu_sc as plsc

NUM_WF = 100000
WLEN = 512
PDIM = 8
NSAMP = 4096

NC = 2
NSUB = 16
NW = NC * NSUB
B_PER_W = NSAMP // NW
CH = 64
NCHUNK = B_PER_W // CH
NBUF = 3

_mesh = plsc.VectorSubcoreMesh(core_axis_name="c", subcore_axis_name="s")
_params = pltpu.CompilerParams(
    needs_layout_passes=False, skip_device_barrier=True)


@functools.partial(
    pl.kernel,
    mesh=_mesh,
    compiler_params=_params,
    out_type=(
        jax.ShapeDtypeStruct((NSAMP, WLEN), jnp.float32),
        jax.ShapeDtypeStruct((NSAMP, WLEN), jnp.float32),
    ),
    scratch_types=[
        pltpu.VMEM((B_PER_W,), jnp.int32),
        [pltpu.VMEM((CH, WLEN), jnp.float32) for _ in range(NBUF)],
        [pltpu.SemaphoreType.DMA for _ in range(NBUF)],
        [pltpu.SemaphoreType.DMA for _ in range(NBUF)],
    ],
)
def _wf_sampler(hplus_hbm, hcross_hbm, idx_hbm, out_p_hbm, out_c_hbm,
                idx_v, rings, gsems, ssems):
    wid = lax.axis_index("s") * NC + lax.axis_index("c")
    base = wid * B_PER_W
    pltpu.sync_copy(idx_hbm.at[pl.ds(base, B_PER_W)], idx_v)

    tables = [hplus_hbm, hcross_hbm]
    outs = [out_p_hbm, out_c_hbm]
    items = [(t, c) for t in range(2) for c in range(NCHUNK)]
    gh = [None] * len(items)
    sh = [None] * len(items)

    def _gather(k):
        t, c = items[k]
        return pltpu.async_copy(
            tables[t].at[idx_v.at[pl.ds(CH * c, CH)]],
            rings[k % NBUF], gsems[k % NBUF])

    def _scatter(k):
        t, c = items[k]
        return pltpu.async_copy(
            rings[k % NBUF], outs[t].at[pl.ds(base + CH * c, CH)],
            ssems[k % NBUF])

    gh[0] = _gather(0)
    for k in range(1, len(items)):
        if k >= NBUF:
            sh[k - NBUF].wait()
        gh[k] = _gather(k)
        gh[k - 1].wait()
        sh[k - 1] = _scatter(k - 1)
    last = len(items) - 1
    gh[last].wait()
    sh[last] = _scatter(last)
    for k in range(max(0, len(items) - NBUF), len(items)):
        sh[k].wait()


@functools.partial(
    pl.kernel,
    mesh=_mesh,
    compiler_params=_params,
    out_type=jax.ShapeDtypeStruct((PDIM, NSAMP), jnp.float32),
    scratch_types=[
        pltpu.VMEM((B_PER_W,), jnp.int32),
        pltpu.VMEM((PDIM, B_PER_W), jnp.float32),
        pltpu.SemaphoreType.DMA,
    ],
)
def _par_sampler(ptflat_hbm, idx_hbm, out_part_hbm, idx_v, par_v, qsem):
    wid = lax.axis_index("s") * NC + lax.axis_index("c")
    base = wid * B_PER_W
    pltpu.sync_copy(idx_hbm.at[pl.ds(base, B_PER_W)], idx_v)
    qhs = []
    for j in range(PDIM):
        qhs.append(pltpu.async_copy(
            ptflat_hbm.at[pl.ds(j * NUM_WF, NUM_WF)].at[idx_v],
            par_v.at[j], qsem))
    for h in qhs:
        h.wait()
    pltpu.sync_copy(par_v, out_part_hbm.at[:, pl.ds(base, B_PER_W)])


def kernel(hplus, hcross, parameters, idx, N):
    idx32 = idx.astype(jnp.int32)
    out_p, out_c = _wf_sampler(hplus, hcross, idx32)
    par_t = _par_sampler(parameters.T.reshape(-1), idx32)
    return (out_p, out_c, par_t.T)


# speedup vs baseline: 1.1636x; 1.1636x over previous
import functools

import jax
import jax.numpy as jnp
from jax import lax
from jax.experimental import pallas as pl
from jax.experimental.pallas import tpu as pltpu
from jax.experimental.pallas import tpu_sc as plsc

NUM_WF = 100000
WLEN = 512
PDIM = 8
NSAMP = 4096

NC = 2
NSUB = 16
NW = NC * NSUB
B_PER_W = NSAMP // NW
CH = 64
NCHUNK = B_PER_W // CH
NBUF = 3

_mesh = plsc.VectorSubcoreMesh(core_axis_name="c", subcore_axis_name="s")


@functools.partial(
    pl.kernel,
    mesh=_mesh,
    compiler_params=pltpu.CompilerParams(
        needs_layout_passes=False, skip_device_barrier=True),
    out_type=(
        jax.ShapeDtypeStruct((NSAMP, WLEN), jnp.float32),
        jax.ShapeDtypeStruct((NSAMP, WLEN), jnp.float32),
        jax.ShapeDtypeStruct((PDIM, NSAMP), jnp.float32),
    ),
    scratch_types=[
        pltpu.VMEM((B_PER_W,), jnp.int32),
        pltpu.VMEM((PDIM, B_PER_W), jnp.float32),
        [pltpu.VMEM((CH, WLEN), jnp.float32) for _ in range(NBUF)],
        [pltpu.SemaphoreType.DMA for _ in range(NBUF)],
        [pltpu.SemaphoreType.DMA for _ in range(NBUF)],
        pltpu.SemaphoreType.DMA,
    ],
)
def _sampler(hplus_hbm, hcross_hbm, ptflat_hbm, idx_hbm,
             out_p_hbm, out_c_hbm, out_part_hbm,
             idx_v, par_v, rings, gsems, ssems, qsem):
    wid = lax.axis_index("s") * NC + lax.axis_index("c")
    base = wid * B_PER_W
    pltpu.sync_copy(idx_hbm.at[pl.ds(base, B_PER_W)], idx_v)

    tables = [hplus_hbm, hcross_hbm]
    outs = [out_p_hbm, out_c_hbm]
    items = [(t, c) for t in range(2) for c in range(NCHUNK)]
    gh = [None] * len(items)
    sh = [None] * len(items)

    def _gather(k):
        t, c = items[k]
        return pltpu.async_copy(
            tables[t].at[idx_v.at[pl.ds(CH * c, CH)]],
            rings[k % NBUF], gsems[k % NBUF])

    def _scatter(k):
        t, c = items[k]
        return pltpu.async_copy(
            rings[k % NBUF], outs[t].at[pl.ds(base + CH * c, CH)],
            ssems[k % NBUF])

    gh[0] = _gather(0)

    qhs = []
    for j in range(PDIM):
        qhs.append(pltpu.async_copy(
            ptflat_hbm.at[pl.ds(j * NUM_WF, NUM_WF)].at[idx_v],
            par_v.at[j], qsem))

    for k in range(1, len(items)):
        if k >= NBUF:
            sh[k - NBUF].wait()
        gh[k] = _gather(k)
        gh[k - 1].wait()
        sh[k - 1] = _scatter(k - 1)
    last = len(items) - 1
    gh[last].wait()
    sh[last] = _scatter(last)
    for k in range(max(0, len(items) - NBUF), len(items)):
        sh[k].wait()

    for h in qhs:
        h.wait()
    pltpu.sync_copy(par_v, out_part_hbm.at[:, pl.ds(base, B_PER_W)])


def kernel(hplus, hcross, parameters, idx, N):
    out_p, out_c, par_t = _sampler(hplus, hcross, parameters.T.reshape(-1),
                                   idx.astype(jnp.int32))
    return (out_p, out_c, par_t.T)
